# baseline (speedup 1.0000x reference)
import jax
import jax.numpy as jnp
from jax.experimental import pallas as pl
from jax.experimental.pallas import tpu as pltpu

_N, _C, _D, _H, _W = 2, 4, 64, 128, 128
_V = _D * _H * _W
_LANES = 128
_ROWS = _V // _LANES
_BR = 512
_G = _ROWS // _BR
_EPS = 0.0001


def _sums_body(x_ref, t_ref, acc_ref):
    g = pl.program_id(1)

    @pl.when(g == 0)
    def _():
        acc_ref[...] = jnp.zeros_like(acc_ref)

    x = x_ref[0]
    t = t_ref[0]
    cidx = jax.lax.broadcasted_iota(jnp.int32, (_C, 1, 1), 0)
    mask = t[None] == cidx
    totals = jnp.sum(x, axis=1)
    sel = jnp.sum(jnp.where(mask, x, 0.0), axis=1)
    cnt = jnp.sum(mask.astype(jnp.float32), axis=1)
    acc_ref[0, 0] += totals
    acc_ref[1, 0] += sel
    acc_ref[2, 0] += cnt


def _dice_sums(x, t):
    xr = x.reshape(_N, _C, _ROWS, _LANES)
    tr = t.reshape(_N, _ROWS, _LANES)
    acc = pl.pallas_call(
        _sums_body,
        grid=(_N, _G),
        in_specs=[
            pl.BlockSpec((1, _C, _BR, _LANES), lambda n, g: (n, 0, g, 0)),
            pl.BlockSpec((1, _BR, _LANES), lambda n, g: (n, g, 0)),
        ],
        out_specs=pl.BlockSpec((3, 1, _C, _LANES), lambda n, g: (0, n, 0, 0)),
        out_shape=jax.ShapeDtypeStruct((3, _N, _C, _LANES), jnp.float32),
    )(xr, tr)
    return acc.sum(-1)


def _dice_loss(sums, weights):
    total, sel, cnt = sums[0], sums[1], sums[2]
    numerator = 2.0 * sel
    denominator = total + cnt + _EPS
    loss_per_channel = weights * (1.0 - numerator / denominator)
    return loss_per_channel.sum() / _N


def kernel(output, gt, shape_output, shape_gt, class_weights):
    out_f = output.reshape(_N, _C, _V)
    sout_f = shape_output.reshape(_N, _C, _V)
    gt_f = gt.reshape(_N, _V).astype(jnp.int32)
    sgt_f = shape_gt.reshape(_N, _V).astype(jnp.int32)
    sums_a = _dice_sums(out_f, gt_f)
    sums_b = _dice_sums(sout_f, sgt_f)
    loss_a = _dice_loss(sums_a, class_weights)
    loss_b = _dice_loss(sums_b, class_weights)
    return (loss_a, loss_b)


# speedup vs baseline: 1.3157x; 1.3157x over previous
import jax
import jax.numpy as jnp
from jax.experimental import pallas as pl
from jax.experimental.pallas import tpu as pltpu

_N, _C, _D, _H, _W = 2, 4, 64, 128, 128
_V = _D * _H * _W
_LANES = 128
_ROWS = _V // _LANES
_BR = 1024
_G = _ROWS // _BR
_EPS = 0.0001


def _sums_body(x_ref, t_ref, acc_ref):
    g = pl.program_id(1)

    @pl.when(g == 0)
    def _():
        acc_ref[...] = jnp.zeros_like(acc_ref)

    x = x_ref[0]
    t = t_ref[0]
    cidx = jax.lax.broadcasted_iota(jnp.int32, (_C, 1, 1), 0)
    mask = t[None] == cidx
    totals = jnp.sum(x, axis=1)
    sel = jnp.sum(jnp.where(mask, x, 0.0), axis=1)
    cnt = jnp.sum(mask.astype(jnp.float32), axis=1)
    acc_ref[0, 0] += totals
    acc_ref[1, 0] += sel
    acc_ref[2, 0] += cnt


def _dice_sums(x, t):
    xr = x.reshape(_N, _C, _ROWS, _LANES)
    tr = t.reshape(_N, _ROWS, _LANES)
    acc = pl.pallas_call(
        _sums_body,
        grid=(_N, _G),
        in_specs=[
            pl.BlockSpec((1, _C, _BR, _LANES), lambda n, g: (n, 0, g, 0)),
            pl.BlockSpec((1, _BR, _LANES), lambda n, g: (n, g, 0)),
        ],
        out_specs=pl.BlockSpec((3, 1, _C, _LANES), lambda n, g: (0, n, 0, 0)),
        out_shape=jax.ShapeDtypeStruct((3, _N, _C, _LANES), jnp.float32),
    )(xr, tr)
    return acc.sum(-1)


def _dice_loss(sums, weights):
    total, sel, cnt = sums[0], sums[1], sums[2]
    numerator = 2.0 * sel
    denominator = total + cnt + _EPS
    loss_per_channel = weights * (1.0 - numerator / denominator)
    return loss_per_channel.sum() / _N


def kernel(output, gt, shape_output, shape_gt, class_weights):
    out_f = output.reshape(_N, _C, _V)
    sout_f = shape_output.reshape(_N, _C, _V)
    gt_f = gt.reshape(_N, _V).astype(jnp.int32)
    sgt_f = shape_gt.reshape(_N, _V).astype(jnp.int32)
    sums_a = _dice_sums(out_f, gt_f)
    sums_b = _dice_sums(sout_f, sgt_f)
    loss_a = _dice_loss(sums_a, class_weights)
    loss_b = _dice_loss(sums_b, class_weights)
    return (loss_a, loss_b)


# speedup vs baseline: 1.5955x; 1.2127x over previous
import jax
import jax.numpy as jnp
from jax.experimental import pallas as pl
from jax.experimental.pallas import tpu as pltpu

_N, _C, _D, _H, _W = 2, 4, 64, 128, 128
_V = _D * _H * _W
_LANES = 128
_ROWS = _V // _LANES
_BR = 2048
_G = _ROWS // _BR
_EPS = 0.0001


def _sums_body(x_ref, t_ref, acc_ref):
    g = pl.program_id(1)

    @pl.when(g == 0)
    def _():
        acc_ref[...] = jnp.zeros_like(acc_ref)

    x = x_ref[0]
    t = t_ref[0]
    cidx = jax.lax.broadcasted_iota(jnp.int32, (_C, 1, 1), 0)
    mask = t[None] == cidx
    totals = jnp.sum(x, axis=1)
    sel = jnp.sum(jnp.where(mask, x, 0.0), axis=1)
    cnt = jnp.sum(mask.astype(jnp.float32), axis=1)
    acc_ref[0, 0] += totals
    acc_ref[1, 0] += sel
    acc_ref[2, 0] += cnt


def _dice_sums(x, t):
    xr = x.reshape(_N, _C, _ROWS, _LANES)
    tr = t.reshape(_N, _ROWS, _LANES)
    acc = pl.pallas_call(
        _sums_body,
        grid=(_N, _G),
        in_specs=[
            pl.BlockSpec((1, _C, _BR, _LANES), lambda n, g: (n, 0, g, 0)),
            pl.BlockSpec((1, _BR, _LANES), lambda n, g: (n, g, 0)),
        ],
        out_specs=pl.BlockSpec((3, 1, _C, _LANES), lambda n, g: (0, n, 0, 0)),
        out_shape=jax.ShapeDtypeStruct((3, _N, _C, _LANES), jnp.float32),
    )(xr, tr)
    return acc.sum(-1)


def _dice_loss(sums, weights):
    total, sel, cnt = sums[0], sums[1], sums[2]
    numerator = 2.0 * sel
    denominator = total + cnt + _EPS
    loss_per_channel = weights * (1.0 - numerator / denominator)
    return loss_per_channel.sum() / _N


def kernel(output, gt, shape_output, shape_gt, class_weights):
    out_f = output.reshape(_N, _C, _V)
    sout_f = shape_output.reshape(_N, _C, _V)
    gt_f = gt.reshape(_N, _V).astype(jnp.int32)
    sgt_f = shape_gt.reshape(_N, _V).astype(jnp.int32)
    sums_a = _dice_sums(out_f, gt_f)
    sums_b = _dice_sums(sout_f, sgt_f)
    loss_a = _dice_loss(sums_a, class_weights)
    loss_b = _dice_loss(sums_b, class_weights)
    return (loss_a, loss_b)


# speedup vs baseline: 1.6636x; 1.0427x over previous
import jax
import jax.numpy as jnp
from jax.experimental import pallas as pl
from jax.experimental.pallas import tpu as pltpu

_N, _C, _D, _H, _W = 2, 4, 64, 128, 128
_V = _D * _H * _W
_LANES = 128
_ROWS = _V // _LANES
_BR = 4096
_G = _ROWS // _BR
_EPS = 0.0001


def _sums_body(x_ref, t_ref, acc_ref):
    g = pl.program_id(1)

    @pl.when(g == 0)
    def _():
        acc_ref[...] = jnp.zeros_like(acc_ref)

    x = x_ref[0]
    t = t_ref[0]
    cidx = jax.lax.broadcasted_iota(jnp.int32, (_C, 1, 1), 0)
    mask = t[None] == cidx
    totals = jnp.sum(x, axis=1)
    sel = jnp.sum(jnp.where(mask, x, 0.0), axis=1)
    cnt = jnp.sum(mask.astype(jnp.float32), axis=1)
    acc_ref[0, 0] += totals
    acc_ref[1, 0] += sel
    acc_ref[2, 0] += cnt


def _dice_sums(x, t):
    xr = x.reshape(_N, _C, _ROWS, _LANES)
    tr = t.reshape(_N, _ROWS, _LANES)
    acc = pl.pallas_call(
        _sums_body,
        grid=(_N, _G),
        in_specs=[
            pl.BlockSpec((1, _C, _BR, _LANES), lambda n, g: (n, 0, g, 0)),
            pl.BlockSpec((1, _BR, _LANES), lambda n, g: (n, g, 0)),
        ],
        out_specs=pl.BlockSpec((3, 1, _C, _LANES), lambda n, g: (0, n, 0, 0)),
        out_shape=jax.ShapeDtypeStruct((3, _N, _C, _LANES), jnp.float32),
    )(xr, tr)
    return acc.sum(-1)


def _dice_loss(sums, weights):
    total, sel, cnt = sums[0], sums[1], sums[2]
    numerator = 2.0 * sel
    denominator = total + cnt + _EPS
    loss_per_channel = weights * (1.0 - numerator / denominator)
    return loss_per_channel.sum() / _N


def kernel(output, gt, shape_output, shape_gt, class_weights):
    out_f = output.reshape(_N, _C, _V)
    sout_f = shape_output.reshape(_N, _C, _V)
    gt_f = gt.reshape(_N, _V).astype(jnp.int32)
    sgt_f = shape_gt.reshape(_N, _V).astype(jnp.int32)
    sums_a = _dice_sums(out_f, gt_f)
    sums_b = _dice_sums(sout_f, sgt_f)
    loss_a = _dice_loss(sums_a, class_weights)
    loss_b = _dice_loss(sums_b, class_weights)
    return (loss_a, loss_b)
